# baseline (speedup 1.0000x reference)
import functools

import jax
import jax.numpy as jnp
from jax import lax
from jax.experimental import pallas as pl
from jax.experimental.pallas import tpu as pltpu
from jax.experimental.pallas import tpu_sc as plsc

_NC = 2
_NS = 16
_L = 16


def _edge_aggregate(table, pk_e, zeros_nt, n_nodes, d):
    CHT, three, K = pk_e.shape
    assert three == 3 and K in (80, 128)
    NB = 2
    NE = 4
    NW = _NC * _NS
    NCH = CHT // NW
    NMAIN = NCH // NE
    n_pad = -(-n_nodes // (_NS * 128)) * (_NS * 128)
    NT = n_pad // _NS
    assert NCH * NW == CHT and NCH % NE == 0 and NMAIN >= 3
    assert zeros_nt.shape == (NT, d) and d % _L == 0 and K % _L == 0

    mesh = plsc.VectorSubcoreMesh(core_axis_name="c", subcore_axis_name="s",
                                  num_cores=_NC, num_subcores=_NS)

    @functools.partial(
        pl.kernel,
        out_type=jax.ShapeDtypeStruct((_NC, n_pad, d), jnp.float32),
        mesh=mesh,
        scratch_types=(
            [pltpu.VMEM((3, K), jnp.int32) for _ in range(NE)]
            + [pltpu.VMEM((K, d), jnp.float32) for _ in range(NB)]
            + [pltpu.VMEM((32, d), jnp.float32)]
            + [pltpu.VMEM_SHARED((n_pad, d), jnp.float32)]
            + [pltpu.SemaphoreType.DMA for _ in range(NB)]
            + [pltpu.SemaphoreType.DMA for _ in range(NB)]
            + [pltpu.SemaphoreType.DMA for _ in range(NE)]
            + [pltpu.SemaphoreType.DMA]
        ),
        compiler_params=pltpu.CompilerParams(needs_layout_passes=False),
    )
    def k(table_h, pk_h, zeros_h, out_h, *refs):
        ebuf = refs[0:NE]
        rows = refs[NE:NE + NB]
        zrows = refs[NE + NB]
        acc = refs[NE + NB + 1]
        gsem = refs[NE + NB + 2:NE + 2 * NB + 2]
        ssem = refs[NE + 2 * NB + 2:NE + 3 * NB + 2]
        esem = refs[NE + 3 * NB + 2:NE + 3 * NB + 2 + NE]
        zsem = refs[NE + 3 * NB + 2 + NE]
        cid = lax.axis_index("c")
        sid = lax.axis_index("s")
        wid = sid * _NC + cid
        cbase = wid * NCH

        def edge_dma(j, be):
            return pltpu.make_async_copy(pk_h.at[cbase + j], ebuf[be], esem[be])

        def gather(be, br):
            return pltpu.make_async_copy(table_h.at[ebuf[be].at[0]], rows[br],
                                         gsem[br])

        def scatter(be, br):
            return pltpu.make_async_copy(rows[br], acc.at[ebuf[be].at[2]],
                                         ssem[br])

        def scale(be, br):
            def sg(g, carry):
                ci16 = ebuf[be][1, pl.ds(g * _L, _L)]
                cs16 = plsc.bitcast(ci16, jnp.float32)
                rb = g * _L
                for il in range(_L):
                    cs = cs16[il]
                    for q in range(d // _L):
                        sl = pl.ds(q * _L, _L)
                        rows[br][rb + il, sl] = rows[br][rb + il, sl] * cs
                return carry
            lax.fori_loop(0, K // _L, sg, 0)

        for j0 in range(NB):
            edge_dma(jnp.int32(j0), j0).start()
        zero16 = jnp.zeros((_L,), jnp.float32)

        def zbody(rr, carry):
            for q in range(d // _L):
                zrows[rr, pl.ds(q * _L, _L)] = zero16
            return carry
        lax.fori_loop(0, 32, zbody, 0)
        for z in range(NT // 32):
            pltpu.sync_copy(zrows, acc.at[pl.ds(sid * NT + z * 32, 32)])
        plsc.subcore_barrier()

        for j0 in range(NB - 1):
            edge_dma(jnp.int32(j0), j0).wait()
            gather(j0, j0).start()

        def substep(j, be, br, first=False, pf_gather=True, pf_edges=True):
            gather(be, br).wait()
            scale(be, br)
            scatter(be, br).start(add=True)
            if not first:
                scatter((be + NE - 1) % NE, (br + NB - 1) % NB).wait()
            if pf_gather:
                beg = (be + NB - 1) % NE
                edge_dma(j + NB - 1, beg).wait()
                gather(beg, (br + NB - 1) % NB).start()
            if pf_edges:
                edge_dma(j + NB, (be + NB) % NE).start()

        def round8(j, first=False, guard=False):
            for i in range(NE):
                jj = j + i
                pg = (not guard) or (NE * (NMAIN - 1) + i + NB - 1 < NCH)
                pe = (not guard) or (NE * (NMAIN - 1) + i + NB < NCH)
                substep(jj, i % NE, i % NB, first=(first and i == 0),
                        pf_gather=pg, pf_edges=pe)

        round8(jnp.int32(0), first=True)
        def main(t, carry):
            round8(t * NE)
            return carry
        lax.fori_loop(1, NMAIN - 1, main, 0)
        round8(jnp.int32(NE * (NMAIN - 1)), guard=True)
        scatter((NCH - 1) % NE, (NCH - 1) % NB).wait()

        plsc.subcore_barrier()
        pltpu.sync_copy(acc.at[pl.ds(sid * NT, NT)],
                        out_h.at[cid, pl.ds(sid * NT, NT)])

    return k(table, pk_e, zeros_nt)


def _edge_pack(src2, rel2, dst2, c2, n_nodes, n_rel, bc):
    CHT, K = src2.shape

    def body(s_ref, r_ref, d_ref, c_ref, p1_ref, p2_ref):
        s = s_ref[...]
        r = r_ref[...]
        dd = d_ref[...]
        ci = jax.lax.bitcast_convert_type(c_ref[...], jnp.int32)
        p1_ref[:, 0, :] = r * n_nodes + s
        p2_ref[:, 0, :] = s * n_rel + r
        p1_ref[:, 1, :] = ci
        p2_ref[:, 1, :] = ci
        p1_ref[:, 2, :] = dd
        p2_ref[:, 2, :] = dd

    return pl.pallas_call(
        body,
        grid=(CHT // bc,),
        in_specs=[
            pl.BlockSpec((bc, K), lambda j: (j, 0)),
            pl.BlockSpec((bc, K), lambda j: (j, 0)),
            pl.BlockSpec((bc, K), lambda j: (j, 0)),
            pl.BlockSpec((bc, K), lambda j: (j, 0)),
        ],
        out_specs=[
            pl.BlockSpec((bc, 3, K), lambda j: (j, 0, 0)),
            pl.BlockSpec((bc, 3, K), lambda j: (j, 0, 0)),
        ],
        out_shape=[
            jax.ShapeDtypeStruct((CHT, 3, K), jnp.int32),
            jax.ShapeDtypeStruct((CHT, 3, K), jnp.int32),
        ],
    )(src2, rel2, dst2, c2)


def _build_table1(comb1, V1, nb):
    B, N, D = V1.shape
    R = comb1.shape[0]

    def body(comb_ref, v1_ref, out_ref):
        v = v1_ref[...]
        for r in range(R):
            acc = comb_ref[r, 0] * v[0]
            for b in range(1, B):
                acc = acc + comb_ref[r, b] * v[b]
            out_ref[r] = acc

    return pl.pallas_call(
        body,
        grid=(N // nb,),
        in_specs=[
            pl.BlockSpec(memory_space=pltpu.SMEM),
            pl.BlockSpec((B, nb, D), lambda j: (0, j, 0)),
        ],
        out_specs=pl.BlockSpec((R, nb, D), lambda j: (0, j, 0)),
        out_shape=jax.ShapeDtypeStruct((R, N, D), jnp.float32),
    )(comb1, V1)


def _layer2_dense(p1, W01, comb2, V2, W02, nb):
    N, D = W01.shape
    R, B = comb2.shape

    def body(comb_ref, p1_ref, w01_ref, v2_ref, w02_ref, xw_ref, hw_ref):
        h = jnp.maximum(p1_ref[0] + p1_ref[1] + w01_ref[...], 0.0)
        v2 = v2_ref[...]
        cats = []
        for r in range(R):
            m = comb_ref[r, 0] * v2[0]
            for b in range(1, B):
                m = m + comb_ref[r, b] * v2[b]
            cats.append(m)
        wcat = jnp.concatenate(cats, axis=1)
        xw_ref[...] = jnp.dot(h, wcat, preferred_element_type=jnp.float32)
        hw_ref[...] = jnp.dot(h, w02_ref[...], preferred_element_type=jnp.float32)

    return pl.pallas_call(
        body,
        grid=(N // nb,),
        in_specs=[
            pl.BlockSpec(memory_space=pltpu.SMEM),
            pl.BlockSpec((2, nb, D), lambda j: (0, j, 0)),
            pl.BlockSpec((nb, D), lambda j: (j, 0)),
            pl.BlockSpec((B, D, D), lambda j: (0, 0, 0)),
            pl.BlockSpec((D, D), lambda j: (0, 0)),
        ],
        out_specs=[
            pl.BlockSpec((nb, R * D), lambda j: (j, 0)),
            pl.BlockSpec((nb, D), lambda j: (j, 0)),
        ],
        out_shape=[
            jax.ShapeDtypeStruct((N, R * D), jnp.float32),
            jax.ShapeDtypeStruct((N, D), jnp.float32),
        ],
    )(comb2, p1, W01, V2, W02)


def _final_out(p2, hw02, nb):
    N, D = hw02.shape

    def body(p2_ref, hw_ref, o_ref):
        o_ref[...] = jnp.maximum(p2_ref[0] + p2_ref[1] + hw_ref[...], 0.0)

    return pl.pallas_call(
        body,
        grid=(N // nb,),
        in_specs=[
            pl.BlockSpec((2, nb, D), lambda j: (0, j, 0)),
            pl.BlockSpec((nb, D), lambda j: (j, 0)),
        ],
        out_specs=pl.BlockSpec((nb, D), lambda j: (j, 0)),
        out_shape=jax.ShapeDtypeStruct((N, D), jnp.float32),
    )(p2, hw02)


def kernel(e_list_true, e_type_true, normc, V1, comb1, W01, V2, comb2, W02):
    B, N, D = V1.shape
    R = comb1.shape[0]
    E = e_list_true.shape[1]
    K = 128
    NW = _NC * _NS

    Ep = -(-E // (NW * K * 4)) * (NW * K * 4)
    pad = Ep - E
    spread = (jnp.arange(pad, dtype=jnp.int32) * 7) % N
    src2 = jnp.concatenate([e_list_true[0].astype(jnp.int32), spread]).reshape(Ep // K, K)
    dst2 = jnp.concatenate([e_list_true[1].astype(jnp.int32), spread]).reshape(Ep // K, K)
    rel2 = jnp.pad(e_type_true[0].astype(jnp.int32), (0, pad)).reshape(Ep // K, K)
    c2 = jnp.pad(normc[0].astype(jnp.float32), (0, pad)).reshape(Ep // K, K)

    pk1, pk2 = _edge_pack(src2, rel2, dst2, c2, N, R, bc=320)

    n_pad = -(-N // (_NS * 128)) * (_NS * 128)
    zeros_nt = jnp.zeros((n_pad // _NS, D), jnp.float32)

    table1 = _build_table1(comb1, V1, nb=1000).reshape(R * N, D)
    p1 = _edge_aggregate(table1, pk1, zeros_nt, n_nodes=N, d=D)

    xwcat, hw02 = _layer2_dense(p1, W01, comb2, V2, W02, nb=1000)
    table2 = xwcat.reshape(N * R, D)

    p2 = _edge_aggregate(table2, pk2, zeros_nt, n_nodes=N, d=D)

    return _final_out(p2, hw02, nb=1000)


# speedup vs baseline: 1.2000x; 1.2000x over previous
import functools

import jax
import jax.numpy as jnp
from jax import lax
from jax.experimental import pallas as pl
from jax.experimental.pallas import tpu as pltpu
from jax.experimental.pallas import tpu_sc as plsc

_NC = 2
_NS = 16
_L = 16


def _edge_aggregate(table, pk_e, zeros_nt, n_nodes, d):
    CHT, three, K = pk_e.shape
    assert three == 3 and K in (80, 128)
    NB = 3
    NE = 6
    NW = _NC * _NS
    NCH = CHT // NW
    NMAIN = NCH // NE
    n_pad = -(-n_nodes // (_NS * 128)) * (_NS * 128)
    NT = n_pad // _NS
    assert NCH * NW == CHT and NCH % NE == 0 and NMAIN >= 3
    assert zeros_nt.shape == (NT, d) and d % _L == 0 and K % _L == 0

    mesh = plsc.VectorSubcoreMesh(core_axis_name="c", subcore_axis_name="s",
                                  num_cores=_NC, num_subcores=_NS)

    @functools.partial(
        pl.kernel,
        out_type=jax.ShapeDtypeStruct((_NC, n_pad, d), jnp.float32),
        mesh=mesh,
        scratch_types=(
            [pltpu.VMEM((3, K), jnp.int32) for _ in range(NE)]
            + [pltpu.VMEM((K, d), jnp.float32) for _ in range(NB)]
            + [pltpu.VMEM((32, d), jnp.float32)]
            + [pltpu.VMEM_SHARED((n_pad, d), jnp.float32)]
            + [pltpu.SemaphoreType.DMA for _ in range(NB)]
            + [pltpu.SemaphoreType.DMA for _ in range(NB)]
            + [pltpu.SemaphoreType.DMA for _ in range(NE)]
            + [pltpu.SemaphoreType.DMA]
        ),
        compiler_params=pltpu.CompilerParams(needs_layout_passes=False),
    )
    def k(table_h, pk_h, zeros_h, out_h, *refs):
        ebuf = refs[0:NE]
        rows = refs[NE:NE + NB]
        zrows = refs[NE + NB]
        acc = refs[NE + NB + 1]
        gsem = refs[NE + NB + 2:NE + 2 * NB + 2]
        ssem = refs[NE + 2 * NB + 2:NE + 3 * NB + 2]
        esem = refs[NE + 3 * NB + 2:NE + 3 * NB + 2 + NE]
        zsem = refs[NE + 3 * NB + 2 + NE]
        cid = lax.axis_index("c")
        sid = lax.axis_index("s")
        wid = sid * _NC + cid
        cbase = wid * NCH

        def edge_dma(j, be):
            return pltpu.make_async_copy(pk_h.at[cbase + j], ebuf[be], esem[be])

        def gather(be, br):
            return pltpu.make_async_copy(table_h.at[ebuf[be].at[0]], rows[br],
                                         gsem[br])

        def scatter(be, br):
            return pltpu.make_async_copy(rows[br], acc.at[ebuf[be].at[2]],
                                         ssem[br])

        def scale(be, br):
            def sg(g, carry):
                ci16 = ebuf[be][1, pl.ds(g * _L, _L)]
                cs16 = plsc.bitcast(ci16, jnp.float32)
                rb = g * _L
                for il in range(_L):
                    cs = cs16[il]
                    for q in range(d // _L):
                        sl = pl.ds(q * _L, _L)
                        rows[br][rb + il, sl] = rows[br][rb + il, sl] * cs
                return carry
            lax.fori_loop(0, K // _L, sg, 0)

        for j0 in range(NB):
            edge_dma(jnp.int32(j0), j0).start()
        zero16 = jnp.zeros((_L,), jnp.float32)

        def zbody(rr, carry):
            for q in range(d // _L):
                zrows[rr, pl.ds(q * _L, _L)] = zero16
            return carry
        lax.fori_loop(0, 32, zbody, 0)
        for z in range(NT // 32):
            pltpu.sync_copy(zrows, acc.at[pl.ds(sid * NT + z * 32, 32)])
        plsc.subcore_barrier()

        for j0 in range(NB - 1):
            edge_dma(jnp.int32(j0), j0).wait()
            gather(j0, j0).start()

        def substep(j, be, br, first=False, pf_gather=True, pf_edges=True):
            gather(be, br).wait()
            scale(be, br)
            scatter(be, br).start(add=True)
            if not first:
                scatter((be + NE - 1) % NE, (br + NB - 1) % NB).wait()
            if pf_gather:
                beg = (be + NB - 1) % NE
                edge_dma(j + NB - 1, beg).wait()
                gather(beg, (br + NB - 1) % NB).start()
            if pf_edges:
                edge_dma(j + NB, (be + NB) % NE).start()

        def round8(j, first=False, guard=False):
            for i in range(NE):
                jj = j + i
                pg = (not guard) or (NE * (NMAIN - 1) + i + NB - 1 < NCH)
                pe = (not guard) or (NE * (NMAIN - 1) + i + NB < NCH)
                substep(jj, i % NE, i % NB, first=(first and i == 0),
                        pf_gather=pg, pf_edges=pe)

        round8(jnp.int32(0), first=True)
        def main(t, carry):
            round8(t * NE)
            return carry
        lax.fori_loop(1, NMAIN - 1, main, 0)
        round8(jnp.int32(NE * (NMAIN - 1)), guard=True)
        scatter((NCH - 1) % NE, (NCH - 1) % NB).wait()

        plsc.subcore_barrier()
        pltpu.sync_copy(acc.at[pl.ds(sid * NT, NT)],
                        out_h.at[cid, pl.ds(sid * NT, NT)])

    return k(table, pk_e, zeros_nt)


def _edge_pack(src2, rel2, dst2, c2, n_nodes, n_rel, bc):
    CHT, K = src2.shape

    def body(s_ref, r_ref, d_ref, c_ref, p1_ref, p2_ref):
        s = s_ref[...]
        r = r_ref[...]
        dd = d_ref[...]
        ci = jax.lax.bitcast_convert_type(c_ref[...], jnp.int32)
        p1_ref[:, 0, :] = r * n_nodes + s
        p2_ref[:, 0, :] = s * n_rel + r
        p1_ref[:, 1, :] = ci
        p2_ref[:, 1, :] = ci
        p1_ref[:, 2, :] = dd
        p2_ref[:, 2, :] = dd

    return pl.pallas_call(
        body,
        grid=(CHT // bc,),
        in_specs=[
            pl.BlockSpec((bc, K), lambda j: (j, 0)),
            pl.BlockSpec((bc, K), lambda j: (j, 0)),
            pl.BlockSpec((bc, K), lambda j: (j, 0)),
            pl.BlockSpec((bc, K), lambda j: (j, 0)),
        ],
        out_specs=[
            pl.BlockSpec((bc, 3, K), lambda j: (j, 0, 0)),
            pl.BlockSpec((bc, 3, K), lambda j: (j, 0, 0)),
        ],
        out_shape=[
            jax.ShapeDtypeStruct((CHT, 3, K), jnp.int32),
            jax.ShapeDtypeStruct((CHT, 3, K), jnp.int32),
        ],
    )(src2, rel2, dst2, c2)


def _build_table1(comb1, V1, nb):
    B, N, D = V1.shape
    R = comb1.shape[0]

    def body(comb_ref, v1_ref, out_ref):
        v = v1_ref[...]
        for r in range(R):
            acc = comb_ref[r, 0] * v[0]
            for b in range(1, B):
                acc = acc + comb_ref[r, b] * v[b]
            out_ref[r] = acc

    return pl.pallas_call(
        body,
        grid=(N // nb,),
        in_specs=[
            pl.BlockSpec(memory_space=pltpu.SMEM),
            pl.BlockSpec((B, nb, D), lambda j: (0, j, 0)),
        ],
        out_specs=pl.BlockSpec((R, nb, D), lambda j: (0, j, 0)),
        out_shape=jax.ShapeDtypeStruct((R, N, D), jnp.float32),
    )(comb1, V1)


def _layer2_dense(p1, W01, comb2, V2, W02, nb):
    N, D = W01.shape
    R, B = comb2.shape

    def body(comb_ref, p1_ref, w01_ref, v2_ref, w02_ref, xw_ref, hw_ref):
        h = jnp.maximum(p1_ref[0] + p1_ref[1] + w01_ref[...], 0.0)
        v2 = v2_ref[...]
        cats = []
        for r in range(R):
            m = comb_ref[r, 0] * v2[0]
            for b in range(1, B):
                m = m + comb_ref[r, b] * v2[b]
            cats.append(m)
        wcat = jnp.concatenate(cats, axis=1)
        xw_ref[...] = jnp.dot(h, wcat, preferred_element_type=jnp.float32)
        hw_ref[...] = jnp.dot(h, w02_ref[...], preferred_element_type=jnp.float32)

    return pl.pallas_call(
        body,
        grid=(N // nb,),
        in_specs=[
            pl.BlockSpec(memory_space=pltpu.SMEM),
            pl.BlockSpec((2, nb, D), lambda j: (0, j, 0)),
            pl.BlockSpec((nb, D), lambda j: (j, 0)),
            pl.BlockSpec((B, D, D), lambda j: (0, 0, 0)),
            pl.BlockSpec((D, D), lambda j: (0, 0)),
        ],
        out_specs=[
            pl.BlockSpec((nb, R * D), lambda j: (j, 0)),
            pl.BlockSpec((nb, D), lambda j: (j, 0)),
        ],
        out_shape=[
            jax.ShapeDtypeStruct((N, R * D), jnp.float32),
            jax.ShapeDtypeStruct((N, D), jnp.float32),
        ],
    )(comb2, p1, W01, V2, W02)


def _final_out(p2, hw02, nb):
    N, D = hw02.shape

    def body(p2_ref, hw_ref, o_ref):
        o_ref[...] = jnp.maximum(p2_ref[0] + p2_ref[1] + hw_ref[...], 0.0)

    return pl.pallas_call(
        body,
        grid=(N // nb,),
        in_specs=[
            pl.BlockSpec((2, nb, D), lambda j: (0, j, 0)),
            pl.BlockSpec((nb, D), lambda j: (j, 0)),
        ],
        out_specs=pl.BlockSpec((nb, D), lambda j: (j, 0)),
        out_shape=jax.ShapeDtypeStruct((N, D), jnp.float32),
    )(p2, hw02)


def kernel(e_list_true, e_type_true, normc, V1, comb1, W01, V2, comb2, W02):
    B, N, D = V1.shape
    R = comb1.shape[0]
    E = e_list_true.shape[1]
    K = 80
    NW = _NC * _NS

    Ep = -(-E // (NW * K * 6)) * (NW * K * 6)
    pad = Ep - E
    spread = (jnp.arange(pad, dtype=jnp.int32) * 7) % N
    src2 = jnp.concatenate([e_list_true[0].astype(jnp.int32), spread]).reshape(Ep // K, K)
    dst2 = jnp.concatenate([e_list_true[1].astype(jnp.int32), spread]).reshape(Ep // K, K)
    rel2 = jnp.pad(e_type_true[0].astype(jnp.int32), (0, pad)).reshape(Ep // K, K)
    c2 = jnp.pad(normc[0].astype(jnp.float32), (0, pad)).reshape(Ep // K, K)

    pk1, pk2 = _edge_pack(src2, rel2, dst2, c2, N, R, bc=504)

    n_pad = -(-N // (_NS * 128)) * (_NS * 128)
    zeros_nt = jnp.zeros((n_pad // _NS, D), jnp.float32)

    table1 = _build_table1(comb1, V1, nb=1000).reshape(R * N, D)
    p1 = _edge_aggregate(table1, pk1, zeros_nt, n_nodes=N, d=D)

    xwcat, hw02 = _layer2_dense(p1, W01, comb2, V2, W02, nb=1000)
    table2 = xwcat.reshape(N * R, D)

    p2 = _edge_aggregate(table2, pk2, zeros_nt, n_nodes=N, d=D)

    return _final_out(p2, hw02, nb=1000)


# speedup vs baseline: 1.2324x; 1.0270x over previous
import functools

import jax
import jax.numpy as jnp
from jax import lax
from jax.experimental import pallas as pl
from jax.experimental.pallas import tpu as pltpu
from jax.experimental.pallas import tpu_sc as plsc

_NC = 2
_NS = 16
_L = 16


def _edge_aggregate(table, pk_e, zeros_nt, n_nodes, d):
    CHT, three, K = pk_e.shape
    assert three == 3 and K in (80, 128)
    NB = 4
    NE = 8
    NW = _NC * _NS
    NCH = CHT // NW
    NMAIN = NCH // NE
    n_pad = -(-n_nodes // (_NS * 128)) * (_NS * 128)
    NT = n_pad // _NS
    assert NCH * NW == CHT and NCH % NE == 0 and NMAIN >= 3
    assert zeros_nt.shape == (NT, d) and d % _L == 0 and K % _L == 0

    mesh = plsc.VectorSubcoreMesh(core_axis_name="c", subcore_axis_name="s",
                                  num_cores=_NC, num_subcores=_NS)

    @functools.partial(
        pl.kernel,
        out_type=jax.ShapeDtypeStruct((_NC, n_pad, d), jnp.float32),
        mesh=mesh,
        scratch_types=(
            [pltpu.VMEM((3, K), jnp.int32) for _ in range(NE)]
            + [pltpu.VMEM((K, d), jnp.float32) for _ in range(NB)]
            + [pltpu.VMEM((32, d), jnp.float32)]
            + [pltpu.VMEM_SHARED((n_pad, d), jnp.float32)]
            + [pltpu.SemaphoreType.DMA for _ in range(NB)]
            + [pltpu.SemaphoreType.DMA for _ in range(NB)]
            + [pltpu.SemaphoreType.DMA for _ in range(NE)]
            + [pltpu.SemaphoreType.DMA]
        ),
        compiler_params=pltpu.CompilerParams(needs_layout_passes=False),
    )
    def k(table_h, pk_h, zeros_h, out_h, *refs):
        ebuf = refs[0:NE]
        rows = refs[NE:NE + NB]
        zrows = refs[NE + NB]
        acc = refs[NE + NB + 1]
        gsem = refs[NE + NB + 2:NE + 2 * NB + 2]
        ssem = refs[NE + 2 * NB + 2:NE + 3 * NB + 2]
        esem = refs[NE + 3 * NB + 2:NE + 3 * NB + 2 + NE]
        zsem = refs[NE + 3 * NB + 2 + NE]
        cid = lax.axis_index("c")
        sid = lax.axis_index("s")
        wid = sid * _NC + cid
        cbase = wid * NCH

        def edge_dma(j, be):
            return pltpu.make_async_copy(pk_h.at[cbase + j], ebuf[be], esem[be])

        def gather(be, br):
            return pltpu.make_async_copy(table_h.at[ebuf[be].at[0]], rows[br],
                                         gsem[br])

        def scatter(be, br):
            return pltpu.make_async_copy(rows[br], acc.at[ebuf[be].at[2]],
                                         ssem[br])

        def scale(be, br):
            def sg(g, carry):
                ci16 = ebuf[be][1, pl.ds(g * _L, _L)]
                cs16 = plsc.bitcast(ci16, jnp.float32)
                rb = g * _L
                for il in range(_L):
                    cs = cs16[il]
                    for q in range(d // _L):
                        sl = pl.ds(q * _L, _L)
                        rows[br][rb + il, sl] = rows[br][rb + il, sl] * cs
                return carry
            lax.fori_loop(0, K // _L, sg, 0)

        for j0 in range(NB):
            edge_dma(jnp.int32(j0), j0).start()
        zero16 = jnp.zeros((_L,), jnp.float32)

        def zbody(rr, carry):
            for q in range(d // _L):
                zrows[rr, pl.ds(q * _L, _L)] = zero16
            return carry
        lax.fori_loop(0, 32, zbody, 0)
        for z in range(NT // 32):
            pltpu.sync_copy(zrows, acc.at[pl.ds(sid * NT + z * 32, 32)])
        plsc.subcore_barrier()

        for j0 in range(NB - 1):
            edge_dma(jnp.int32(j0), j0).wait()
            gather(j0, j0).start()

        def substep(j, be, br, first=False, pf_gather=True, pf_edges=True):
            gather(be, br).wait()
            scale(be, br)
            scatter(be, br).start(add=True)
            if not first:
                scatter((be + NE - 1) % NE, (br + NB - 1) % NB).wait()
            if pf_gather:
                beg = (be + NB - 1) % NE
                edge_dma(j + NB - 1, beg).wait()
                gather(beg, (br + NB - 1) % NB).start()
            if pf_edges:
                edge_dma(j + NB, (be + NB) % NE).start()

        def round8(j, first=False, guard=False):
            for i in range(NE):
                jj = j + i
                pg = (not guard) or (NE * (NMAIN - 1) + i + NB - 1 < NCH)
                pe = (not guard) or (NE * (NMAIN - 1) + i + NB < NCH)
                substep(jj, i % NE, i % NB, first=(first and i == 0),
                        pf_gather=pg, pf_edges=pe)

        round8(jnp.int32(0), first=True)
        def main(t, carry):
            round8(t * NE)
            return carry
        lax.fori_loop(1, NMAIN - 1, main, 0)
        round8(jnp.int32(NE * (NMAIN - 1)), guard=True)
        scatter((NCH - 1) % NE, (NCH - 1) % NB).wait()

        plsc.subcore_barrier()
        pltpu.sync_copy(acc.at[pl.ds(sid * NT, NT)],
                        out_h.at[cid, pl.ds(sid * NT, NT)])

    return k(table, pk_e, zeros_nt)


def _edge_pack(src2, rel2, dst2, c2, n_nodes, n_rel, bc):
    CHT, K = src2.shape

    def body(s_ref, r_ref, d_ref, c_ref, p1_ref, p2_ref):
        s = s_ref[...]
        r = r_ref[...]
        dd = d_ref[...]
        ci = jax.lax.bitcast_convert_type(c_ref[...], jnp.int32)
        p1_ref[:, 0, :] = r * n_nodes + s
        p2_ref[:, 0, :] = s * n_rel + r
        p1_ref[:, 1, :] = ci
        p2_ref[:, 1, :] = ci
        p1_ref[:, 2, :] = dd
        p2_ref[:, 2, :] = dd

    return pl.pallas_call(
        body,
        grid=(CHT // bc,),
        in_specs=[
            pl.BlockSpec((bc, K), lambda j: (j, 0)),
            pl.BlockSpec((bc, K), lambda j: (j, 0)),
            pl.BlockSpec((bc, K), lambda j: (j, 0)),
            pl.BlockSpec((bc, K), lambda j: (j, 0)),
        ],
        out_specs=[
            pl.BlockSpec((bc, 3, K), lambda j: (j, 0, 0)),
            pl.BlockSpec((bc, 3, K), lambda j: (j, 0, 0)),
        ],
        out_shape=[
            jax.ShapeDtypeStruct((CHT, 3, K), jnp.int32),
            jax.ShapeDtypeStruct((CHT, 3, K), jnp.int32),
        ],
    )(src2, rel2, dst2, c2)


def _build_table1(comb1, V1, nb):
    B, N, D = V1.shape
    R = comb1.shape[0]

    def body(comb_ref, v1_ref, out_ref):
        v = v1_ref[...]
        for r in range(R):
            acc = comb_ref[r, 0] * v[0]
            for b in range(1, B):
                acc = acc + comb_ref[r, b] * v[b]
            out_ref[r] = acc

    return pl.pallas_call(
        body,
        grid=(N // nb,),
        in_specs=[
            pl.BlockSpec(memory_space=pltpu.SMEM),
            pl.BlockSpec((B, nb, D), lambda j: (0, j, 0)),
        ],
        out_specs=pl.BlockSpec((R, nb, D), lambda j: (0, j, 0)),
        out_shape=jax.ShapeDtypeStruct((R, N, D), jnp.float32),
    )(comb1, V1)


def _layer2_dense(p1, W01, comb2, V2, W02, nb):
    N, D = W01.shape
    R, B = comb2.shape

    def body(comb_ref, p1_ref, w01_ref, v2_ref, w02_ref, xw_ref, hw_ref):
        h = jnp.maximum(p1_ref[0] + p1_ref[1] + w01_ref[...], 0.0)
        v2 = v2_ref[...]
        cats = []
        for r in range(R):
            m = comb_ref[r, 0] * v2[0]
            for b in range(1, B):
                m = m + comb_ref[r, b] * v2[b]
            cats.append(m)
        wcat = jnp.concatenate(cats, axis=1)
        xw_ref[...] = jnp.dot(h, wcat, preferred_element_type=jnp.float32)
        hw_ref[...] = jnp.dot(h, w02_ref[...], preferred_element_type=jnp.float32)

    return pl.pallas_call(
        body,
        grid=(N // nb,),
        in_specs=[
            pl.BlockSpec(memory_space=pltpu.SMEM),
            pl.BlockSpec((2, nb, D), lambda j: (0, j, 0)),
            pl.BlockSpec((nb, D), lambda j: (j, 0)),
            pl.BlockSpec((B, D, D), lambda j: (0, 0, 0)),
            pl.BlockSpec((D, D), lambda j: (0, 0)),
        ],
        out_specs=[
            pl.BlockSpec((nb, R * D), lambda j: (j, 0)),
            pl.BlockSpec((nb, D), lambda j: (j, 0)),
        ],
        out_shape=[
            jax.ShapeDtypeStruct((N, R * D), jnp.float32),
            jax.ShapeDtypeStruct((N, D), jnp.float32),
        ],
    )(comb2, p1, W01, V2, W02)


def _final_out(p2, hw02, nb):
    N, D = hw02.shape

    def body(p2_ref, hw_ref, o_ref):
        o_ref[...] = jnp.maximum(p2_ref[0] + p2_ref[1] + hw_ref[...], 0.0)

    return pl.pallas_call(
        body,
        grid=(N // nb,),
        in_specs=[
            pl.BlockSpec((2, nb, D), lambda j: (0, j, 0)),
            pl.BlockSpec((nb, D), lambda j: (j, 0)),
        ],
        out_specs=pl.BlockSpec((nb, D), lambda j: (j, 0)),
        out_shape=jax.ShapeDtypeStruct((N, D), jnp.float32),
    )(p2, hw02)


def kernel(e_list_true, e_type_true, normc, V1, comb1, W01, V2, comb2, W02):
    B, N, D = V1.shape
    R = comb1.shape[0]
    E = e_list_true.shape[1]
    K = 80
    NW = _NC * _NS

    Ep = -(-E // (NW * K * 8)) * (NW * K * 8)
    pad = Ep - E
    spread = (jnp.arange(pad, dtype=jnp.int32) * 7) % N
    src2 = jnp.concatenate([e_list_true[0].astype(jnp.int32), spread]).reshape(Ep // K, K)
    dst2 = jnp.concatenate([e_list_true[1].astype(jnp.int32), spread]).reshape(Ep // K, K)
    rel2 = jnp.pad(e_type_true[0].astype(jnp.int32), (0, pad)).reshape(Ep // K, K)
    c2 = jnp.pad(normc[0].astype(jnp.float32), (0, pad)).reshape(Ep // K, K)

    pk1, pk2 = _edge_pack(src2, rel2, dst2, c2, N, R, bc=512)

    n_pad = -(-N // (_NS * 128)) * (_NS * 128)
    zeros_nt = jnp.zeros((n_pad // _NS, D), jnp.float32)

    table1 = _build_table1(comb1, V1, nb=1000).reshape(R * N, D)
    p1 = _edge_aggregate(table1, pk1, zeros_nt, n_nodes=N, d=D)

    xwcat, hw02 = _layer2_dense(p1, W01, comb2, V2, W02, nb=1000)
    table2 = xwcat.reshape(N * R, D)

    p2 = _edge_aggregate(table2, pk2, zeros_nt, n_nodes=N, d=D)

    return _final_out(p2, hw02, nb=1000)


# speedup vs baseline: 1.2517x; 1.0156x over previous
import functools

import jax
import jax.numpy as jnp
from jax import lax
from jax.experimental import pallas as pl
from jax.experimental.pallas import tpu as pltpu
from jax.experimental.pallas import tpu_sc as plsc

_NC = 2
_NS = 16
_L = 16


def _edge_aggregate(table, pk_e, zeros_nt, n_nodes, d):
    CHT, three, K = pk_e.shape
    assert three == 3 and K in (80, 128)
    NB = 4
    NE = 8
    NW = _NC * _NS
    NCH = CHT // NW
    NMAIN = NCH // NE
    n_pad = -(-n_nodes // (_NS * 128)) * (_NS * 128)
    NT = n_pad // _NS
    assert NCH * NW == CHT and NCH % NE == 0 and NMAIN >= 3
    assert zeros_nt.shape == (NT, d) and d % _L == 0 and K % _L == 0

    mesh = plsc.VectorSubcoreMesh(core_axis_name="c", subcore_axis_name="s",
                                  num_cores=_NC, num_subcores=_NS)

    @functools.partial(
        pl.kernel,
        out_type=jax.ShapeDtypeStruct((_NC, n_pad, d), jnp.float32),
        mesh=mesh,
        scratch_types=(
            [pltpu.VMEM((3, K), jnp.int32) for _ in range(NE)]
            + [pltpu.VMEM((K, d), jnp.float32) for _ in range(NB)]
            + [pltpu.VMEM((32, d), jnp.float32)]
            + [pltpu.VMEM_SHARED((n_pad, d), jnp.float32)]
            + [pltpu.SemaphoreType.DMA for _ in range(NB)]
            + [pltpu.SemaphoreType.DMA for _ in range(NB)]
            + [pltpu.SemaphoreType.DMA for _ in range(NE)]
            + [pltpu.SemaphoreType.DMA]
        ),
        compiler_params=pltpu.CompilerParams(needs_layout_passes=False),
    )
    def k(table_h, pk_h, zeros_h, out_h, *refs):
        ebuf = refs[0:NE]
        rows = refs[NE:NE + NB]
        zrows = refs[NE + NB]
        acc = refs[NE + NB + 1]
        gsem = refs[NE + NB + 2:NE + 2 * NB + 2]
        ssem = refs[NE + 2 * NB + 2:NE + 3 * NB + 2]
        esem = refs[NE + 3 * NB + 2:NE + 3 * NB + 2 + NE]
        zsem = refs[NE + 3 * NB + 2 + NE]
        cid = lax.axis_index("c")
        sid = lax.axis_index("s")
        wid = sid * _NC + cid
        cbase = wid * NCH

        def edge_dma(j, be):
            return pltpu.make_async_copy(pk_h.at[cbase + j], ebuf[be], esem[be])

        def gather(be, br):
            return pltpu.make_async_copy(table_h.at[ebuf[be].at[0]], rows[br],
                                         gsem[br])

        def scatter(be, br):
            return pltpu.make_async_copy(rows[br], acc.at[ebuf[be].at[2]],
                                         ssem[br])

        def scale(be, br):
            def sg(g, carry):
                ci16 = ebuf[be][1, pl.ds(g * _L, _L)]
                cs16 = plsc.bitcast(ci16, jnp.float32)
                rb = g * _L
                for il in range(_L):
                    cs = cs16[il]
                    for q in range(d // _L):
                        sl = pl.ds(q * _L, _L)
                        rows[br][rb + il, sl] = rows[br][rb + il, sl] * cs
                return carry
            lax.fori_loop(0, K // _L, sg, 0)

        for j0 in range(NB):
            edge_dma(jnp.int32(j0), j0).start()
        zero16 = jnp.zeros((_L,), jnp.float32)

        def zbody(rr, carry):
            for q in range(d // _L):
                zrows[rr, pl.ds(q * _L, _L)] = zero16
            return carry
        lax.fori_loop(0, 32, zbody, 0)
        for z in range(NT // 32):
            pltpu.sync_copy(zrows, acc.at[pl.ds(sid * NT + z * 32, 32)])
        plsc.subcore_barrier()

        for j0 in range(NB - 1):
            edge_dma(jnp.int32(j0), j0).wait()
            gather(j0, j0).start()

        def substep(j, be, br, first=False, pf_gather=True, pf_edges=True):
            gather(be, br).wait()
            scale(be, br)
            scatter(be, br).start(add=True)
            if not first:
                scatter((be + NE - 1) % NE, (br + NB - 1) % NB).wait()
            if pf_gather:
                beg = (be + NB - 1) % NE
                edge_dma(j + NB - 1, beg).wait()
                gather(beg, (br + NB - 1) % NB).start()
            if pf_edges:
                edge_dma(j + NB, (be + NB) % NE).start()

        def round8(j, first=False, guard=False):
            for i in range(NE):
                jj = j + i
                pg = (not guard) or (NE * (NMAIN - 1) + i + NB - 1 < NCH)
                pe = (not guard) or (NE * (NMAIN - 1) + i + NB < NCH)
                substep(jj, i % NE, i % NB, first=(first and i == 0),
                        pf_gather=pg, pf_edges=pe)

        round8(jnp.int32(0), first=True)
        def main(t, carry):
            round8(t * NE)
            return carry
        lax.fori_loop(1, NMAIN - 1, main, 0)
        round8(jnp.int32(NE * (NMAIN - 1)), guard=True)
        scatter((NCH - 1) % NE, (NCH - 1) % NB).wait()

        plsc.subcore_barrier()
        pltpu.sync_copy(acc.at[pl.ds(sid * NT, NT)],
                        out_h.at[cid, pl.ds(sid * NT, NT)])

    return k(table, pk_e, zeros_nt)


def _edge_pack(src2, rel2, dst2, c2, n_nodes, n_rel, bc):
    CHT, K = src2.shape

    def body(s_ref, r_ref, d_ref, c_ref, p1_ref, p2_ref):
        s = s_ref[...]
        r = r_ref[...]
        dd = d_ref[...]
        ci = jax.lax.bitcast_convert_type(c_ref[...], jnp.int32)
        p1_ref[:, 0, :] = r * n_nodes + s
        p2_ref[:, 0, :] = s * n_rel + r
        p1_ref[:, 1, :] = ci
        p2_ref[:, 1, :] = ci
        p1_ref[:, 2, :] = dd
        p2_ref[:, 2, :] = dd

    return pl.pallas_call(
        body,
        grid=(CHT // bc,),
        in_specs=[
            pl.BlockSpec((bc, K), lambda j: (j, 0)),
            pl.BlockSpec((bc, K), lambda j: (j, 0)),
            pl.BlockSpec((bc, K), lambda j: (j, 0)),
            pl.BlockSpec((bc, K), lambda j: (j, 0)),
        ],
        out_specs=[
            pl.BlockSpec((bc, 3, K), lambda j: (j, 0, 0)),
            pl.BlockSpec((bc, 3, K), lambda j: (j, 0, 0)),
        ],
        out_shape=[
            jax.ShapeDtypeStruct((CHT, 3, K), jnp.int32),
            jax.ShapeDtypeStruct((CHT, 3, K), jnp.int32),
        ],
    )(src2, rel2, dst2, c2)


def _build_table1(comb1, V1, nb):
    B, N, D = V1.shape
    R = comb1.shape[0]

    def body(comb_ref, v1_ref, out_ref):
        v = v1_ref[...]
        for r in range(R):
            acc = comb_ref[r, 0] * v[0]
            for b in range(1, B):
                acc = acc + comb_ref[r, b] * v[b]
            out_ref[r] = acc

    return pl.pallas_call(
        body,
        grid=(N // nb,),
        in_specs=[
            pl.BlockSpec(memory_space=pltpu.SMEM),
            pl.BlockSpec((B, nb, D), lambda j: (0, j, 0)),
        ],
        out_specs=pl.BlockSpec((R, nb, D), lambda j: (0, j, 0)),
        out_shape=jax.ShapeDtypeStruct((R, N, D), jnp.float32),
    )(comb1, V1)


def _layer2_dense(p1, W01, comb2, V2, W02, nb):
    N, D = W01.shape
    R, B = comb2.shape

    def body(comb_ref, p1_ref, w01_ref, v2_ref, w02_ref, xw_ref, hw_ref):
        h = jnp.maximum(p1_ref[0] + p1_ref[1] + w01_ref[...], 0.0)
        v2 = v2_ref[...]
        cats = []
        for r in range(R):
            m = comb_ref[r, 0] * v2[0]
            for b in range(1, B):
                m = m + comb_ref[r, b] * v2[b]
            cats.append(m)
        wcat = jnp.concatenate(cats, axis=1)
        xw_ref[...] = jnp.dot(h, wcat, preferred_element_type=jnp.float32)
        hw_ref[...] = jnp.dot(h, w02_ref[...], preferred_element_type=jnp.float32)

    return pl.pallas_call(
        body,
        grid=(N // nb,),
        in_specs=[
            pl.BlockSpec(memory_space=pltpu.SMEM),
            pl.BlockSpec((2, nb, D), lambda j: (0, j, 0)),
            pl.BlockSpec((nb, D), lambda j: (j, 0)),
            pl.BlockSpec((B, D, D), lambda j: (0, 0, 0)),
            pl.BlockSpec((D, D), lambda j: (0, 0)),
        ],
        out_specs=[
            pl.BlockSpec((nb, R * D), lambda j: (j, 0)),
            pl.BlockSpec((nb, D), lambda j: (j, 0)),
        ],
        out_shape=[
            jax.ShapeDtypeStruct((N, R * D), jnp.float32),
            jax.ShapeDtypeStruct((N, D), jnp.float32),
        ],
    )(comb2, p1, W01, V2, W02)


def _final_out(p2, hw02, nb):
    N, D = hw02.shape

    def body(p2_ref, hw_ref, o_ref):
        o_ref[...] = jnp.maximum(p2_ref[0] + p2_ref[1] + hw_ref[...], 0.0)

    return pl.pallas_call(
        body,
        grid=(N // nb,),
        in_specs=[
            pl.BlockSpec((2, nb, D), lambda j: (0, j, 0)),
            pl.BlockSpec((nb, D), lambda j: (j, 0)),
        ],
        out_specs=pl.BlockSpec((nb, D), lambda j: (j, 0)),
        out_shape=jax.ShapeDtypeStruct((N, D), jnp.float32),
    )(p2, hw02)


def kernel(e_list_true, e_type_true, normc, V1, comb1, W01, V2, comb2, W02):
    B, N, D = V1.shape
    R = comb1.shape[0]
    E = e_list_true.shape[1]
    K = 80
    NW = _NC * _NS

    Ep = -(-E // (NW * K * 8)) * (NW * K * 8)
    pad = Ep - E
    spread = (jnp.arange(pad, dtype=jnp.int32) * 7) % N
    src2 = jnp.concatenate([e_list_true[0].astype(jnp.int32), spread]).reshape(Ep // K, K)
    dst2 = jnp.concatenate([e_list_true[1].astype(jnp.int32), spread]).reshape(Ep // K, K)
    rel2 = jnp.pad(e_type_true[0].astype(jnp.int32), (0, pad)).reshape(Ep // K, K)
    c2 = jnp.pad(normc[0].astype(jnp.float32), (0, pad)).reshape(Ep // K, K)

    pk1, pk2 = _edge_pack(src2, rel2, dst2, c2, N, R, bc=1024)

    n_pad = -(-N // (_NS * 128)) * (_NS * 128)
    zeros_nt = jnp.zeros((n_pad // _NS, D), jnp.float32)

    table1 = _build_table1(comb1, V1, nb=2000).reshape(R * N, D)
    p1 = _edge_aggregate(table1, pk1, zeros_nt, n_nodes=N, d=D)

    xwcat, hw02 = _layer2_dense(p1, W01, comb2, V2, W02, nb=2000)
    table2 = xwcat.reshape(N * R, D)

    p2 = _edge_aggregate(table2, pk2, zeros_nt, n_nodes=N, d=D)

    return _final_out(p2, hw02, nb=2000)


# speedup vs baseline: 1.2527x; 1.0008x over previous
import functools

import jax
import jax.numpy as jnp
from jax import lax
from jax.experimental import pallas as pl
from jax.experimental.pallas import tpu as pltpu
from jax.experimental.pallas import tpu_sc as plsc

_NC = 2
_NS = 16
_L = 16


def _edge_aggregate(table, pk_e, n_nodes, d):
    CHT, three, K = pk_e.shape
    assert three == 3 and K in (80, 128)
    NB = 4
    NE = 8
    NW = _NC * _NS
    NCH = CHT // NW
    NMAIN = NCH // NE
    n_pad = -(-n_nodes // (_NS * 128)) * (_NS * 128)
    NT = n_pad // _NS
    assert NCH * NW == CHT and NCH % NE == 0 and NMAIN >= 3
    assert d % _L == 0 and K % _L == 0

    mesh = plsc.VectorSubcoreMesh(core_axis_name="c", subcore_axis_name="s",
                                  num_cores=_NC, num_subcores=_NS)

    @functools.partial(
        pl.kernel,
        out_type=jax.ShapeDtypeStruct((_NC, n_pad, d), jnp.float32),
        mesh=mesh,
        scratch_types=(
            [pltpu.VMEM((3, K), jnp.int32) for _ in range(NE)]
            + [pltpu.VMEM((K, d), jnp.float32) for _ in range(NB)]
            + [pltpu.VMEM((32, d), jnp.float32)]
            + [pltpu.VMEM_SHARED((n_pad, d), jnp.float32)]
            + [pltpu.SemaphoreType.DMA for _ in range(NB)]
            + [pltpu.SemaphoreType.DMA for _ in range(NB)]
            + [pltpu.SemaphoreType.DMA for _ in range(NE)]
        ),
        compiler_params=pltpu.CompilerParams(needs_layout_passes=False),
    )
    def k(table_h, pk_h, out_h, *refs):
        ebuf = refs[0:NE]
        rows = refs[NE:NE + NB]
        zrows = refs[NE + NB]
        acc = refs[NE + NB + 1]
        gsem = refs[NE + NB + 2:NE + 2 * NB + 2]
        ssem = refs[NE + 2 * NB + 2:NE + 3 * NB + 2]
        esem = refs[NE + 3 * NB + 2:NE + 3 * NB + 2 + NE]
        cid = lax.axis_index("c")
        sid = lax.axis_index("s")
        wid = sid * _NC + cid
        cbase = wid * NCH

        def edge_dma(j, be):
            return pltpu.make_async_copy(pk_h.at[cbase + j], ebuf[be], esem[be])

        def gather(be, br):
            return pltpu.make_async_copy(table_h.at[ebuf[be].at[0]], rows[br],
                                         gsem[br])

        def scatter(be, br):
            return pltpu.make_async_copy(rows[br], acc.at[ebuf[be].at[2]],
                                         ssem[br])

        def scale(be, br):
            def sg(g, carry):
                ci16 = ebuf[be][1, pl.ds(g * _L, _L)]
                cs16 = plsc.bitcast(ci16, jnp.float32)
                rb = g * _L
                for il in range(_L):
                    cs = cs16[il]
                    for q in range(d // _L):
                        sl = pl.ds(q * _L, _L)
                        rows[br][rb + il, sl] = rows[br][rb + il, sl] * cs
                return carry
            lax.fori_loop(0, K // _L, sg, 0)

        for j0 in range(NB):
            edge_dma(jnp.int32(j0), j0).start()
        zero16 = jnp.zeros((_L,), jnp.float32)

        def zbody(rr, carry):
            for q in range(d // _L):
                zrows[rr, pl.ds(q * _L, _L)] = zero16
            return carry
        lax.fori_loop(0, 32, zbody, 0)
        for z in range(NT // 32):
            pltpu.sync_copy(zrows, acc.at[pl.ds(sid * NT + z * 32, 32)])
        plsc.subcore_barrier()

        for j0 in range(NB - 1):
            edge_dma(jnp.int32(j0), j0).wait()
            gather(j0, j0).start()

        def substep(j, be, br, first=False, pf_gather=True, pf_edges=True):
            gather(be, br).wait()
            scale(be, br)
            scatter(be, br).start(add=True)
            if not first:
                scatter((be + NE - 1) % NE, (br + NB - 1) % NB).wait()
            if pf_gather:
                beg = (be + NB - 1) % NE
                edge_dma(j + NB - 1, beg).wait()
                gather(beg, (br + NB - 1) % NB).start()
            if pf_edges:
                edge_dma(j + NB, (be + NB) % NE).start()

        def round8(j, first=False, guard=False):
            for i in range(NE):
                jj = j + i
                pg = (not guard) or (NE * (NMAIN - 1) + i + NB - 1 < NCH)
                pe = (not guard) or (NE * (NMAIN - 1) + i + NB < NCH)
                substep(jj, i % NE, i % NB, first=(first and i == 0),
                        pf_gather=pg, pf_edges=pe)

        round8(jnp.int32(0), first=True)
        def main(t, carry):
            round8(t * NE)
            return carry
        lax.fori_loop(1, NMAIN - 1, main, 0)
        round8(jnp.int32(NE * (NMAIN - 1)), guard=True)
        scatter((NCH - 1) % NE, (NCH - 1) % NB).wait()

        plsc.subcore_barrier()
        pltpu.sync_copy(acc.at[pl.ds(sid * NT, NT)],
                        out_h.at[cid, pl.ds(sid * NT, NT)])

    return k(table, pk_e)


def _edge_pack(src2, rel2, dst2, c2, n_nodes, n_rel, bc):
    CHT, K = src2.shape

    def body(s_ref, r_ref, d_ref, c_ref, p1_ref, p2_ref):
        s = s_ref[...]
        r = r_ref[...]
        dd = d_ref[...]
        ci = jax.lax.bitcast_convert_type(c_ref[...], jnp.int32)
        p1_ref[:, 0, :] = r * n_nodes + s
        p2_ref[:, 0, :] = s * n_rel + r
        p1_ref[:, 1, :] = ci
        p2_ref[:, 1, :] = ci
        p1_ref[:, 2, :] = dd
        p2_ref[:, 2, :] = dd

    return pl.pallas_call(
        body,
        grid=(CHT // bc,),
        in_specs=[
            pl.BlockSpec((bc, K), lambda j: (j, 0)),
            pl.BlockSpec((bc, K), lambda j: (j, 0)),
            pl.BlockSpec((bc, K), lambda j: (j, 0)),
            pl.BlockSpec((bc, K), lambda j: (j, 0)),
        ],
        out_specs=[
            pl.BlockSpec((bc, 3, K), lambda j: (j, 0, 0)),
            pl.BlockSpec((bc, 3, K), lambda j: (j, 0, 0)),
        ],
        out_shape=[
            jax.ShapeDtypeStruct((CHT, 3, K), jnp.int32),
            jax.ShapeDtypeStruct((CHT, 3, K), jnp.int32),
        ],
    )(src2, rel2, dst2, c2)


def _build_table1(comb1, V1, nb):
    B, N, D = V1.shape
    R = comb1.shape[0]

    def body(comb_ref, v1_ref, out_ref):
        v = v1_ref[...]
        for r in range(R):
            acc = comb_ref[r, 0] * v[0]
            for b in range(1, B):
                acc = acc + comb_ref[r, b] * v[b]
            out_ref[r] = acc

    return pl.pallas_call(
        body,
        grid=(N // nb,),
        in_specs=[
            pl.BlockSpec(memory_space=pltpu.SMEM),
            pl.BlockSpec((B, nb, D), lambda j: (0, j, 0)),
        ],
        out_specs=pl.BlockSpec((R, nb, D), lambda j: (0, j, 0)),
        out_shape=jax.ShapeDtypeStruct((R, N, D), jnp.float32),
    )(comb1, V1)


def _layer2_dense(p1, W01, comb2, V2, W02, nb):
    N, D = W01.shape
    R, B = comb2.shape

    def body(comb_ref, p1_ref, w01_ref, v2_ref, w02_ref, xw_ref, hw_ref):
        h = jnp.maximum(p1_ref[0] + p1_ref[1] + w01_ref[...], 0.0)
        v2 = v2_ref[...]
        cats = []
        for r in range(R):
            m = comb_ref[r, 0] * v2[0]
            for b in range(1, B):
                m = m + comb_ref[r, b] * v2[b]
            cats.append(m)
        wcat = jnp.concatenate(cats, axis=1)
        xw_ref[...] = jnp.dot(h, wcat, preferred_element_type=jnp.float32)
        hw_ref[...] = jnp.dot(h, w02_ref[...], preferred_element_type=jnp.float32)

    return pl.pallas_call(
        body,
        grid=(N // nb,),
        in_specs=[
            pl.BlockSpec(memory_space=pltpu.SMEM),
            pl.BlockSpec((2, nb, D), lambda j: (0, j, 0)),
            pl.BlockSpec((nb, D), lambda j: (j, 0)),
            pl.BlockSpec((B, D, D), lambda j: (0, 0, 0)),
            pl.BlockSpec((D, D), lambda j: (0, 0)),
        ],
        out_specs=[
            pl.BlockSpec((nb, R * D), lambda j: (j, 0)),
            pl.BlockSpec((nb, D), lambda j: (j, 0)),
        ],
        out_shape=[
            jax.ShapeDtypeStruct((N, R * D), jnp.float32),
            jax.ShapeDtypeStruct((N, D), jnp.float32),
        ],
    )(comb2, p1, W01, V2, W02)


def _final_out(p2, hw02, nb):
    N, D = hw02.shape

    def body(p2_ref, hw_ref, o_ref):
        o_ref[...] = jnp.maximum(p2_ref[0] + p2_ref[1] + hw_ref[...], 0.0)

    return pl.pallas_call(
        body,
        grid=(N // nb,),
        in_specs=[
            pl.BlockSpec((2, nb, D), lambda j: (0, j, 0)),
            pl.BlockSpec((nb, D), lambda j: (j, 0)),
        ],
        out_specs=pl.BlockSpec((nb, D), lambda j: (j, 0)),
        out_shape=jax.ShapeDtypeStruct((N, D), jnp.float32),
    )(p2, hw02)


def kernel(e_list_true, e_type_true, normc, V1, comb1, W01, V2, comb2, W02):
    B, N, D = V1.shape
    R = comb1.shape[0]
    E = e_list_true.shape[1]
    K = 80
    NW = _NC * _NS

    Ep = -(-E // (NW * K * 8)) * (NW * K * 8)
    pad = Ep - E
    spread = (jnp.arange(pad, dtype=jnp.int32) * 7) % N
    src2 = jnp.concatenate([e_list_true[0].astype(jnp.int32), spread]).reshape(Ep // K, K)
    dst2 = jnp.concatenate([e_list_true[1].astype(jnp.int32), spread]).reshape(Ep // K, K)
    rel2 = jnp.pad(e_type_true[0].astype(jnp.int32), (0, pad)).reshape(Ep // K, K)
    c2 = jnp.pad(normc[0].astype(jnp.float32), (0, pad)).reshape(Ep // K, K)

    pk1, pk2 = _edge_pack(src2, rel2, dst2, c2, N, R, bc=1024)


    table1 = _build_table1(comb1, V1, nb=2000).reshape(R * N, D)
    p1 = _edge_aggregate(table1, pk1, n_nodes=N, d=D)

    xwcat, hw02 = _layer2_dense(p1, W01, comb2, V2, W02, nb=2000)
    table2 = xwcat.reshape(N * R, D)

    p2 = _edge_aggregate(table2, pk2, n_nodes=N, d=D)

    return _final_out(p2, hw02, nb=2000)
